# Initial kernel scaffold; baseline (speedup 1.0000x reference)
import jax
import jax.numpy as jnp
from jax.experimental import pallas as pl


def kernel(x, W_gate, W_noise):
    raise NotImplementedError("write your pallas kernel here")


# speedup vs baseline: 6.6780x; 6.6780x over previous
import functools

import jax
import jax.numpy as jnp
from jax import lax
from jax.experimental import pallas as pl
from jax.experimental.pallas import tpu as pltpu
from jax.experimental.pallas import tpu_sc as plsc

DIM = 768
NUM_EXPERTS = 8
TOP_K = 2

_NC = 2
_NS = 16
_NW = _NC * _NS
_L = 16


def _logits_body(x_ref, w_ref, o_ref):
    o_ref[...] = lax.dot_general(
        w_ref[...], x_ref[...],
        (((1,), (1,)), ((), ())),
        preferred_element_type=jnp.float32,
    )


def _logits_tc(x2, w_gate, bt):
    n = x2.shape[0]
    grid = (n // bt,)
    return pl.pallas_call(
        _logits_body,
        grid=grid,
        in_specs=[
            pl.BlockSpec((bt, DIM), lambda i: (i, 0)),
            pl.BlockSpec((NUM_EXPERTS, DIM), lambda i: (0, 0)),
        ],
        out_specs=pl.BlockSpec((NUM_EXPERTS, bt), lambda i: (0, i)),
        out_shape=jax.ShapeDtypeStruct((NUM_EXPERTS, n), jnp.float32),
    )(x2, w_gate)


def _make_route(n):
    tpw = n // _NW
    groups = tpw // _L
    mesh = plsc.VectorSubcoreMesh(core_axis_name="c", subcore_axis_name="s")

    @functools.partial(
        pl.kernel,
        mesh=mesh,
        out_type=[
            jax.ShapeDtypeStruct((n,), jnp.int32),
            jax.ShapeDtypeStruct((n,), jnp.int32),
            jax.ShapeDtypeStruct((n,), jnp.float32),
            jax.ShapeDtypeStruct((n,), jnp.float32),
        ],
        scratch_types=[
            pltpu.VMEM((NUM_EXPERTS, tpw), jnp.float32),
            pltpu.VMEM((tpw,), jnp.int32),
            pltpu.VMEM((tpw,), jnp.int32),
            pltpu.VMEM((tpw,), jnp.float32),
            pltpu.VMEM((tpw,), jnp.float32),
        ],
    )
    def route(lt_hbm, i1_hbm, i2_hbm, w1_hbm, w2_hbm, lv, i1v, i2v, w1v, w2v):
        wid = lax.axis_index("c") * _NS + lax.axis_index("s")
        base = wid * tpw
        pltpu.sync_copy(lt_hbm.at[:, pl.ds(base, tpw)], lv)

        neg = jnp.full((_L,), -jnp.inf, jnp.float32)
        zero_i = jnp.zeros((_L,), jnp.int32)

        def body(g, carry):
            off = g * _L
            m1, m2, i1, i2 = neg, neg, zero_i, zero_i
            for e in range(NUM_EXPERTS):
                v = lv[e, pl.ds(off, _L)]
                ev = jnp.full((_L,), e, jnp.int32)
                gt1 = v > m1
                gt2 = v > m2
                i2 = jnp.where(gt1, i1, jnp.where(gt2, ev, i2))
                m2 = jnp.where(gt1, m1, jnp.where(gt2, v, m2))
                i1 = jnp.where(gt1, ev, i1)
                m1 = jnp.where(gt1, v, m1)
            ed = jnp.exp(m2 - m1)
            denom = 1.0 + ed
            w1 = 1.0 / denom
            w2 = ed / denom
            i1v[pl.ds(off, _L)] = i1
            i2v[pl.ds(off, _L)] = i2
            w1v[pl.ds(off, _L)] = w1
            w2v[pl.ds(off, _L)] = w2
            return carry

        lax.fori_loop(0, groups, body, 0)
        pltpu.sync_copy(i1v, i1_hbm.at[pl.ds(base, tpw)])
        pltpu.sync_copy(i2v, i2_hbm.at[pl.ds(base, tpw)])
        pltpu.sync_copy(w1v, w1_hbm.at[pl.ds(base, tpw)])
        pltpu.sync_copy(w2v, w2_hbm.at[pl.ds(base, tpw)])

    return route


def kernel(x, W_gate, W_noise):
    orig_shape = x.shape
    x2 = x.reshape(-1, orig_shape[-1])
    n = x2.shape[0]
    lt = _logits_tc(x2, W_gate, 2048)
    i1, i2, w1, w2 = _make_route(n)(lt)
    leading = orig_shape[:-1]
    topi = jnp.stack([i1, i2], axis=-1).reshape(*leading, TOP_K)
    weights = jnp.stack([w1, w2], axis=-1).reshape(*leading, TOP_K)
    priority = w1.reshape(leading)
    return topi, weights, priority


# speedup vs baseline: 6.7134x; 1.0053x over previous
import functools

import jax
import jax.numpy as jnp
from jax import lax
from jax.experimental import pallas as pl
from jax.experimental.pallas import tpu as pltpu
from jax.experimental.pallas import tpu_sc as plsc

DIM = 768
NUM_EXPERTS = 8
TOP_K = 2

_NC = 2
_NS = 16
_NW = _NC * _NS
_L = 16


def _logits_body(x_ref, w_ref, o_ref):
    o_ref[...] = lax.dot_general(
        w_ref[...], x_ref[...],
        (((1,), (1,)), ((), ())),
        preferred_element_type=jnp.float32,
    )


def _logits_tc(x2, w_gate, bt):
    n = x2.shape[0]
    grid = (n // bt,)
    return pl.pallas_call(
        _logits_body,
        grid=grid,
        in_specs=[
            pl.BlockSpec((bt, DIM), lambda i: (i, 0)),
            pl.BlockSpec((NUM_EXPERTS, DIM), lambda i: (0, 0)),
        ],
        out_specs=pl.BlockSpec((NUM_EXPERTS, bt), lambda i: (0, i)),
        out_shape=jax.ShapeDtypeStruct((NUM_EXPERTS, n), jnp.float32),
    )(x2, w_gate)


def _make_route(n):
    tpw = n // _NW
    groups = tpw // _L
    mesh = plsc.VectorSubcoreMesh(core_axis_name="c", subcore_axis_name="s")

    @functools.partial(
        pl.kernel,
        mesh=mesh,
        out_type=[
            jax.ShapeDtypeStruct((n,), jnp.int32),
            jax.ShapeDtypeStruct((n,), jnp.int32),
            jax.ShapeDtypeStruct((n,), jnp.float32),
            jax.ShapeDtypeStruct((n,), jnp.float32),
        ],
        scratch_types=[
            pltpu.VMEM((NUM_EXPERTS, tpw), jnp.float32),
            pltpu.VMEM((tpw,), jnp.int32),
            pltpu.VMEM((tpw,), jnp.int32),
            pltpu.VMEM((tpw,), jnp.float32),
            pltpu.VMEM((tpw,), jnp.float32),
        ],
    )
    def route(lt_hbm, i1_hbm, i2_hbm, w1_hbm, w2_hbm, lv, i1v, i2v, w1v, w2v):
        wid = lax.axis_index("c") * _NS + lax.axis_index("s")
        base = wid * tpw
        pltpu.sync_copy(lt_hbm.at[:, pl.ds(base, tpw)], lv)

        neg = jnp.full((_L,), -jnp.inf, jnp.float32)
        zero_i = jnp.zeros((_L,), jnp.int32)

        def body(g, carry):
            off = g * _L
            m1, m2, i1, i2 = neg, neg, zero_i, zero_i
            for e in range(NUM_EXPERTS):
                v = lv[e, pl.ds(off, _L)]
                ev = jnp.full((_L,), e, jnp.int32)
                gt1 = v > m1
                gt2 = v > m2
                i2 = jnp.where(gt1, i1, jnp.where(gt2, ev, i2))
                m2 = jnp.where(gt1, m1, jnp.where(gt2, v, m2))
                i1 = jnp.where(gt1, ev, i1)
                m1 = jnp.where(gt1, v, m1)
            ed = jnp.exp(m2 - m1)
            denom = 1.0 + ed
            w1 = 1.0 / denom
            w2 = ed / denom
            i1v[pl.ds(off, _L)] = i1
            i2v[pl.ds(off, _L)] = i2
            w1v[pl.ds(off, _L)] = w1
            w2v[pl.ds(off, _L)] = w2
            return carry

        lax.fori_loop(0, groups, body, 0)
        pltpu.sync_copy(i1v, i1_hbm.at[pl.ds(base, tpw)])
        pltpu.sync_copy(i2v, i2_hbm.at[pl.ds(base, tpw)])
        pltpu.sync_copy(w1v, w1_hbm.at[pl.ds(base, tpw)])
        pltpu.sync_copy(w2v, w2_hbm.at[pl.ds(base, tpw)])

    return route


def kernel(x, W_gate, W_noise):
    orig_shape = x.shape
    x2 = x.reshape(-1, orig_shape[-1])
    n = x2.shape[0]
    lt = _logits_tc(x2, W_gate, 4096)
    i1, i2, w1, w2 = _make_route(n)(lt)
    leading = orig_shape[:-1]
    topi = jnp.stack([i1, i2], axis=-1).reshape(*leading, TOP_K)
    weights = jnp.stack([w1, w2], axis=-1).reshape(*leading, TOP_K)
    priority = w1.reshape(leading)
    return topi, weights, priority
